# baseline (speedup 1.0000x reference)
import functools

import jax
import jax.numpy as jnp
from jax import lax
from jax.experimental import pallas as pl
from jax.experimental.pallas import tpu as pltpu
from jax.experimental.pallas import tpu_sc as plsc

N = 100000
NP = 102400
EH = 100000
EHP = 102400
EPT = EHP // 32
G = 96
GPAD = 128
CAPB = 6400
DUMP = NP
_SC_MESH = dict(core_axis_name="c", subcore_axis_name="s")


def _scalar_lane(vec, i):
    return jnp.sum(jnp.where(lax.iota(jnp.int32, 16) == i, vec, jnp.int32(0)))


def _uv_body(heT, proj_hbm, u_hbm, v_hbm, proj_v, e_vs, u_v, v_v):
    wid = lax.axis_index("c") * 16 + lax.axis_index("s")
    base = wid * EPT
    pltpu.sync_copy(proj_hbm.at[pl.ds(0, N)], proj_v)
    for k in range(4):
        pltpu.sync_copy(heT.at[pl.ds(k * EHP + base, EPT)], e_vs[k])

    def body(i, carry):
        off = i * 16
        e0 = e_vs[0][pl.ds(off, 16)]
        p0 = plsc.load_gather(proj_v, [e0])
        ubest, pmax = e0, p0
        vbest, pmin = e0, p0
        for k in range(1, 4):
            ek = e_vs[k][pl.ds(off, 16)]
            pk = plsc.load_gather(proj_v, [ek])
            mx = pk > pmax
            ubest = jnp.where(mx, ek, ubest)
            pmax = jnp.where(mx, pk, pmax)
            mn = pk < pmin
            vbest = jnp.where(mn, ek, vbest)
            pmin = jnp.where(mn, pk, pmin)
        valid = (base + off + lax.iota(jnp.int32, 16)) < EH
        dumpv = DUMP + lax.iota(jnp.int32, 16)
        u_v[pl.ds(off, 16)] = jnp.where(valid, ubest, dumpv)
        v_v[pl.ds(off, 16)] = jnp.where(valid, vbest, dumpv)
        return carry

    lax.fori_loop(0, EPT // 16, body, 0)
    pltpu.sync_copy(u_v, u_hbm.at[pl.ds(base, EPT)])
    pltpu.sync_copy(v_v, v_hbm.at[pl.ds(base, EPT)])


@functools.partial(
    pl.kernel,
    out_type=[
        jax.ShapeDtypeStruct((EHP,), jnp.int32),
        jax.ShapeDtypeStruct((EHP,), jnp.int32),
    ],
    mesh=plsc.VectorSubcoreMesh(**_SC_MESH),
    scratch_types=[
        pltpu.VMEM((N,), jnp.float32),
        [pltpu.VMEM((EPT,), jnp.int32)] * 4,
        pltpu.VMEM((EPT,), jnp.int32),
        pltpu.VMEM((EPT,), jnp.int32),
    ],
    compiler_params=pltpu.CompilerParams(needs_layout_passes=False, use_tc_tiling_on_sc=False),
)
def _uv_kernel(heT, proj, u_out, v_out, proj_v, e_vs, u_v, v_v):
    _uv_body(heT, proj, u_out, v_out, proj_v, e_vs, u_v, v_v)


def _make_bin_kernel(nchunk, chrows):
    def body(u_hbm, v_hbm, bd_hbm, bs_hbm, nb_hbm, u_v, v_v, bd_vs, bs_vs, nb_v):
        wid = lax.axis_index("c") * 16 + lax.axis_index("s")
        base = wid * EPT
        pltpu.sync_copy(u_hbm.at[pl.ds(base, EPT)], u_v)
        pltpu.sync_copy(v_hbm.at[pl.ds(base, EPT)], v_v)
        iot = lax.iota(jnp.int32, 16)

        def it(i, cnts):
            off = i * 16
            uu = u_v[pl.ds(off, 16)]
            vv = v_v[pl.ds(off, 16)]
            cnts = list(cnts)
            for dd, ss in ((vv, uu), (uu, vv)):
                for c in range(nchunk):
                    lo = c * chrows
                    m = (dd >= lo) & (dd < lo + chrows)
                    mi = m.astype(jnp.int32)
                    incl = plsc.cumsum(mi)
                    pos = cnts[c] + incl - mi
                    plsc.store_scatter(bd_vs[c], [pos], dd - lo, mask=m)
                    plsc.store_scatter(bs_vs[c], [pos], ss, mask=m)
                    cnts[c] = cnts[c] + jnp.max(incl)
            return tuple(cnts)

        cnts = lax.fori_loop(0, EPT // 16, it, (jnp.int32(0),) * nchunk)
        nbvec = jnp.zeros((16,), jnp.int32)
        for c in range(nchunk):
            for k in range(GPAD // 16):
                pos = cnts[c] + k * 16 + iot
                plsc.store_scatter(bd_vs[c], [pos], chrows + iot)
                plsc.store_scatter(bs_vs[c], [pos], wid * GPAD + k * 16 + iot)
            nbvec = jnp.where(iot == c, cnts[c], nbvec)
        nb_v[pl.ds(0, 16)] = nbvec
        pltpu.sync_copy(nb_v, nb_hbm.at[pl.ds(wid * 16, 16)])
        for c in range(nchunk):
            pltpu.sync_copy(bd_vs[c], bd_hbm.at[pl.ds((c * 32 + wid) * CAPB, CAPB)])
            pltpu.sync_copy(bs_vs[c], bs_hbm.at[pl.ds((c * 32 + wid) * CAPB, CAPB)])

    return pl.kernel(
        body,
        out_type=[
            jax.ShapeDtypeStruct((nchunk * 32 * CAPB,), jnp.int32),
            jax.ShapeDtypeStruct((nchunk * 32 * CAPB,), jnp.int32),
            jax.ShapeDtypeStruct((512,), jnp.int32),
        ],
        mesh=plsc.VectorSubcoreMesh(**_SC_MESH),
        scratch_types=[
            pltpu.VMEM((EPT,), jnp.int32),
            pltpu.VMEM((EPT,), jnp.int32),
            [pltpu.VMEM((CAPB,), jnp.int32)] * nchunk,
            [pltpu.VMEM((CAPB,), jnp.int32)] * nchunk,
            pltpu.VMEM((16,), jnp.int32),
        ],
        compiler_params=pltpu.CompilerParams(needs_layout_passes=False, use_tc_tiling_on_sc=False),
    )


def _histlen(chrows):
    return -(-(chrows + 16) // 256) * 256


DHIST = 102656
_DSTRIPE = DHIST // 16


def _deg_body(u2_hbm, v2_hbm, ones_hbm, zeros_hbm, cnt_hbm, hist_sh, ub2, vb2, ones_b, sem):
    cid = lax.axis_index("c")
    sid = lax.axis_index("s")
    wid = cid * 16 + sid
    rows = EPT // 128
    pltpu.sync_copy(zeros_hbm.at[pl.ds(sid * _DSTRIPE, _DSTRIPE)],
                    hist_sh.at[pl.ds(sid * _DSTRIPE, _DSTRIPE)])
    pltpu.sync_copy(ones_hbm, ones_b)
    pltpu.sync_copy(u2_hbm.at[pl.ds(wid * rows, rows)], ub2)
    pltpu.sync_copy(v2_hbm.at[pl.ds(wid * rows, rows)], vb2)
    plsc.subcore_barrier()
    hs = []
    for b in range(rows):
        hs.append(pltpu.async_copy(ones_b.at[b], hist_sh.at[ub2.at[b]], sem, add=True))
        hs.append(pltpu.async_copy(ones_b.at[b], hist_sh.at[vb2.at[b]], sem, add=True))
    for h in hs:
        h.wait()
    plsc.subcore_barrier()
    pltpu.sync_copy(hist_sh.at[pl.ds(sid * _DSTRIPE, _DSTRIPE)],
                    cnt_hbm.at[pl.ds(cid * DHIST + sid * _DSTRIPE, _DSTRIPE)])


_deg_kernel = pl.kernel(
    _deg_body,
    out_type=jax.ShapeDtypeStruct((2 * DHIST,), jnp.float32),
    mesh=plsc.VectorSubcoreMesh(**_SC_MESH),
    scratch_types=[
        pltpu.VMEM_SHARED((DHIST,), jnp.float32),
        pltpu.VMEM((EPT // 128, 128), jnp.int32),
        pltpu.VMEM((EPT // 128, 128), jnp.int32),
        pltpu.VMEM((EPT // 128, 128), jnp.float32),
        pltpu.SemaphoreType.DMA,
    ],
    compiler_params=pltpu.CompilerParams(needs_layout_passes=False, use_tc_tiling_on_sc=False),
)


def _make_prep_kernel(nchunk, chrows):
    rows = EPT // 128

    def body(heT, proj_hbm, ones_hbm, zeros_hbm,
             bd_hbm, bs_hbm, nb_hbm, cnt_hbm,
             u2, v2, nb_v, hist_sh, ones_b, sem):
        cid = lax.axis_index("c")
        sid = lax.axis_index("s")
        wid = cid * 16 + sid
        base = wid * EPT
        iot = lax.iota(jnp.int32, 16)

        def phase_a(proj_v, e_vs):
            pltpu.sync_copy(proj_hbm.at[pl.ds(0, N)], proj_v)
            for k in range(4):
                pltpu.sync_copy(heT.at[pl.ds(k * EHP + base, EPT)], e_vs[k])

            def rowit(r, carry):
                for k in range(8):
                    off = r * 128 + k * 16
                    e0 = e_vs[0][pl.ds(off, 16)]
                    p0 = plsc.load_gather(proj_v, [e0])
                    ubest, pmax = e0, p0
                    vbest, pmin = e0, p0
                    for q in range(1, 4):
                        eq = e_vs[q][pl.ds(off, 16)]
                        pq = plsc.load_gather(proj_v, [eq])
                        mx = pq > pmax
                        ubest = jnp.where(mx, eq, ubest)
                        pmax = jnp.where(mx, pq, pmax)
                        mn = pq < pmin
                        vbest = jnp.where(mn, eq, vbest)
                        pmin = jnp.where(mn, pq, pmin)
                    valid = (base + off + iot) < EH
                    dumpv = DUMP + iot
                    u2[r, pl.ds(k * 16, 16)] = jnp.where(valid, ubest, dumpv)
                    v2[r, pl.ds(k * 16, 16)] = jnp.where(valid, vbest, dumpv)
                return carry

            lax.fori_loop(0, rows, rowit, 0)

        pl.run_scoped(
            phase_a,
            pltpu.VMEM((N,), jnp.float32),
            [pltpu.VMEM((EPT,), jnp.int32)] * 4,
        )

        def phase_b(bd_vs, bs_vs):
            def rowit(r, cnts):
                cnts = list(cnts)
                for k in range(8):
                    uu = u2[r, pl.ds(k * 16, 16)]
                    vv = v2[r, pl.ds(k * 16, 16)]
                    for dd, ss in ((vv, uu), (uu, vv)):
                        for c in range(nchunk):
                            lo = c * chrows
                            m = (dd >= lo) & (dd < lo + chrows)
                            mi = m.astype(jnp.int32)
                            incl = plsc.cumsum(mi)
                            pos = cnts[c] + incl - mi
                            plsc.store_scatter(bd_vs[c], [pos], dd - lo, mask=m)
                            plsc.store_scatter(bs_vs[c], [pos], ss, mask=m)
                            cnts[c] = cnts[c] + jnp.max(incl)
                return tuple(cnts)

            cnts = lax.fori_loop(0, rows, rowit, (jnp.int32(0),) * nchunk)
            nbvec = jnp.zeros((16,), jnp.int32)
            for c in range(nchunk):
                for k in range(GPAD // 16):
                    pos = cnts[c] + k * 16 + iot
                    plsc.store_scatter(bd_vs[c], [pos], chrows + iot)
                    plsc.store_scatter(bs_vs[c], [pos], wid * GPAD + k * 16 + iot)
                nbvec = jnp.where(iot == c, cnts[c], nbvec)
            nb_v[pl.ds(0, 16)] = nbvec
            pltpu.sync_copy(nb_v, nb_hbm.at[pl.ds(wid * 16, 16)])
            for c in range(nchunk):
                pltpu.sync_copy(bd_vs[c], bd_hbm.at[pl.ds((c * 32 + wid) * CAPB, CAPB)])
                pltpu.sync_copy(bs_vs[c], bs_hbm.at[pl.ds((c * 32 + wid) * CAPB, CAPB)])

        pl.run_scoped(
            phase_b,
            [pltpu.VMEM((CAPB,), jnp.int32)] * nchunk,
            [pltpu.VMEM((CAPB,), jnp.int32)] * nchunk,
        )

        pltpu.sync_copy(zeros_hbm.at[pl.ds(sid * _DSTRIPE, _DSTRIPE)],
                        hist_sh.at[pl.ds(sid * _DSTRIPE, _DSTRIPE)])
        pltpu.sync_copy(ones_hbm, ones_b)
        plsc.subcore_barrier()
        hs = []
        for b in range(rows):
            hs.append(pltpu.async_copy(ones_b.at[b], hist_sh.at[u2.at[b]], sem, add=True))
            hs.append(pltpu.async_copy(ones_b.at[b], hist_sh.at[v2.at[b]], sem, add=True))
        for h in hs:
            h.wait()
        plsc.subcore_barrier()
        pltpu.sync_copy(hist_sh.at[pl.ds(sid * _DSTRIPE, _DSTRIPE)],
                        cnt_hbm.at[pl.ds(cid * DHIST + sid * _DSTRIPE, _DSTRIPE)])

    return pl.kernel(
        body,
        out_type=[
            jax.ShapeDtypeStruct((nchunk * 32 * CAPB,), jnp.int32),
            jax.ShapeDtypeStruct((nchunk * 32 * CAPB,), jnp.int32),
            jax.ShapeDtypeStruct((512,), jnp.int32),
            jax.ShapeDtypeStruct((2 * DHIST,), jnp.float32),
        ],
        mesh=plsc.VectorSubcoreMesh(**_SC_MESH),
        scratch_types=[
            pltpu.VMEM((EPT // 128, 128), jnp.int32),
            pltpu.VMEM((EPT // 128, 128), jnp.int32),
            pltpu.VMEM((16,), jnp.int32),
            pltpu.VMEM_SHARED((DHIST,), jnp.float32),
            pltpu.VMEM((EPT // 128, 128), jnp.float32),
            pltpu.SemaphoreType.DMA,
        ],
        compiler_params=pltpu.CompilerParams(needs_layout_passes=False, use_tc_tiling_on_sc=False),
    )


def _make_row_kernel(nchunk, chrows, cpc, roww, tc_tiling=False):
    accr = chrows + 16
    sr = chrows // 16

    def body(y_hbm, bd_hbm, bs_hbm, nb_hbm, zp_hbm, acc_sh,
             db3, sb3, rows3, cbuf, sg):
        cid = lax.axis_index("c")
        sid = lax.axis_index("s")

        def seg_loop(c, w):
            pltpu.sync_copy(nb_hbm.at[pl.ds(w * 16, 16)], cbuf)
            cnt = _scalar_lane(cbuf[pl.ds(0, 16)], c)
            seg = (c * 32 + w) * CAPB

            def step(b, carry):
                par = lax.rem(b, 2)

                @pl.when(b * G < cnt)
                def _():
                    pltpu.sync_copy(bd_hbm.at[pl.ds(seg + b * G, G)], db3.at[par])
                    pltpu.sync_copy(bs_hbm.at[pl.ds(seg + b * G, G)], sb3.at[par])
                    pltpu.async_copy(y_hbm.at[sb3.at[par]], rows3.at[par], sg)

                @pl.when((b > 0) & ((b - 1) * G < cnt))
                def _():
                    q = 1 - par
                    pltpu.make_async_copy(y_hbm.at[sb3.at[q]], rows3.at[q], sg).wait()
                    pltpu.sync_copy(rows3.at[q], acc_sh.at[db3.at[q]], add=True)

                return carry

            lax.fori_loop(0, lax.shift_right_logical(cnt, 6) + 2, step, 0)

        for kk in range(cpc):
            c = cid * cpc + kk
            pltpu.sync_copy(y_hbm.at[pl.ds(c * chrows + sid * sr, sr)],
                            acc_sh.at[pl.ds(sid * sr, sr)])
            plsc.subcore_barrier()
            for j in range(2):
                seg_loop(c, 2 * sid + j)
            plsc.subcore_barrier()
            pltpu.sync_copy(acc_sh.at[pl.ds(sid * sr, sr)],
                            zp_hbm.at[pl.ds(c * chrows + sid * sr, sr)])

    return pl.kernel(
        body,
        out_type=jax.ShapeDtypeStruct((NP, roww), jnp.float32),
        mesh=plsc.VectorSubcoreMesh(**_SC_MESH),
        scratch_types=[
            pltpu.VMEM_SHARED((accr, roww), jnp.float32),
            pltpu.VMEM((2, G), jnp.int32),
            pltpu.VMEM((2, G), jnp.int32),
            pltpu.VMEM((2, G, roww), jnp.float32),
            pltpu.VMEM((16,), jnp.int32),
            pltpu.SemaphoreType.DMA,
        ],
        compiler_params=pltpu.CompilerParams(
            needs_layout_passes=False, use_tc_tiling_on_sc=tc_tiling),
    )


def _mm_proj_body(x_ref, W_ref, rv_ref, X_ref, p_ref):
    X = jnp.dot(x_ref[...], W_ref[...], preferred_element_type=jnp.float32)
    X_ref[...] = X
    p_ref[...] = jnp.dot(X, rv_ref[...], preferred_element_type=jnp.float32)


def _matmul_proj(x, W, rv, Bn=800):
    n, d = x.shape
    h = W.shape[1]
    return pl.pallas_call(
        _mm_proj_body,
        grid=(n // Bn,),
        in_specs=[
            pl.BlockSpec((Bn, d), lambda i: (i, 0)),
            pl.BlockSpec((d, h), lambda i: (0, 0)),
            pl.BlockSpec((h, 1), lambda i: (0, 0)),
        ],
        out_specs=[
            pl.BlockSpec((Bn, h), lambda i: (i, 0)),
            pl.BlockSpec((Bn, 1), lambda i: (i, 0)),
        ],
        out_shape=[
            jax.ShapeDtypeStruct((NP, h), jnp.float32),
            jax.ShapeDtypeStruct((NP, 1), jnp.float32),
        ],
    )(x, W, rv)


def _scale_body(x_ref, ca_ref, cb_ref, y_ref, dis_ref):
    dis = lax.rsqrt(jnp.maximum(ca_ref[...] + cb_ref[...] + 1.0, 1e-12))
    dis_ref[...] = dis
    y_ref[...] = x_ref[...] * dis


def _mm_scale_body(x_ref, W_ref, ca_ref, cb_ref, y_ref, dis_ref):
    dis = lax.rsqrt(jnp.maximum(ca_ref[...] + cb_ref[...] + 1.0, 1e-12))
    dis_ref[...] = dis
    X = jnp.dot(x_ref[...], W_ref[...], preferred_element_type=jnp.float32)
    y_ref[...] = X * dis


def _mm_scale(x, W, ca, cb, Bn=800):
    n, d = x.shape
    h = W.shape[1]
    return pl.pallas_call(
        _mm_scale_body,
        grid=(n // Bn,),
        in_specs=[
            pl.BlockSpec((Bn, d), lambda i: (i, 0)),
            pl.BlockSpec((d, h), lambda i: (0, 0)),
            pl.BlockSpec((Bn, 1), lambda i: (i, 0)),
            pl.BlockSpec((Bn, 1), lambda i: (i, 0)),
        ],
        out_specs=[
            pl.BlockSpec((Bn, h), lambda i: (i, 0)),
            pl.BlockSpec((Bn, 1), lambda i: (i, 0)),
        ],
        out_shape=[
            jax.ShapeDtypeStruct((NP, h), jnp.float32),
            jax.ShapeDtypeStruct((NP, 1), jnp.float32),
        ],
    )(x, W, ca, cb)


def _make_scale_kernel(roww, Bn=2048):
    return pl.pallas_call(
        _scale_body,
        grid=(NP // Bn,),
        in_specs=[
            pl.BlockSpec((Bn, roww), lambda i: (i, 0)),
            pl.BlockSpec((Bn, 1), lambda i: (i, 0)),
            pl.BlockSpec((Bn, 1), lambda i: (i, 0)),
        ],
        out_specs=[
            pl.BlockSpec((Bn, roww), lambda i: (i, 0)),
            pl.BlockSpec((Bn, 1), lambda i: (i, 0)),
        ],
        out_shape=[
            jax.ShapeDtypeStruct((NP, roww), jnp.float32),
            jax.ShapeDtypeStruct((NP, 1), jnp.float32),
        ],
    )


def _bn_mm_body(zp_ref, dis_ref, g_ref, b_ref, W_ref, rv_ref, x2_ref, p2_ref):
    t = jnp.maximum(zp_ref[...] * dis_ref[...], 0.0)
    t = t * g_ref[...] + b_ref[...]
    X2 = jnp.dot(t, W_ref[...], preferred_element_type=jnp.float32)
    x2_ref[...] = X2
    p2_ref[...] = jnp.dot(X2, rv_ref[...], preferred_element_type=jnp.float32)


def _bn_mm(zp, dis, g2d, b2d, W2, rv2, Bn=2048):
    h, c = W2.shape
    return pl.pallas_call(
        _bn_mm_body,
        grid=(NP // Bn,),
        in_specs=[
            pl.BlockSpec((Bn, h), lambda i: (i, 0)),
            pl.BlockSpec((Bn, 1), lambda i: (i, 0)),
            pl.BlockSpec((1, h), lambda i: (0, 0)),
            pl.BlockSpec((1, h), lambda i: (0, 0)),
            pl.BlockSpec((h, c), lambda i: (0, 0)),
            pl.BlockSpec((c, 1), lambda i: (0, 0)),
        ],
        out_specs=[
            pl.BlockSpec((Bn, c), lambda i: (i, 0)),
            pl.BlockSpec((Bn, 1), lambda i: (i, 0)),
        ],
        out_shape=[
            jax.ShapeDtypeStruct((NP, c), jnp.float32),
            jax.ShapeDtypeStruct((NP, 1), jnp.float32),
        ],
    )(zp, dis, g2d, b2d, W2, rv2)


def _lsm_body(zp_ref, dis_ref, out_ref):
    L = zp_ref[...] * dis_ref[...]
    m = jnp.max(L, axis=-1, keepdims=True)
    s = jnp.log(jnp.sum(jnp.exp(L - m), axis=-1, keepdims=True))
    out_ref[...] = L - m - s


def _lsm(zp, dis, c, Bn=800):
    return pl.pallas_call(
        _lsm_body,
        grid=(N // Bn,),
        in_specs=[
            pl.BlockSpec((Bn, c), lambda i: (i, 0)),
            pl.BlockSpec((Bn, 1), lambda i: (i, 0)),
        ],
        out_specs=pl.BlockSpec((Bn, c), lambda i: (i, 0)),
        out_shape=jax.ShapeDtypeStruct((N, c), jnp.float32),
    )(zp, dis)


_L1 = dict(nchunk=8, chrows=12800, cpc=4)
_L2 = dict(nchunk=4, chrows=25600, cpc=2)
_PREP1 = _make_prep_kernel(_L1["nchunk"], _L1["chrows"])
_PREP2 = _make_prep_kernel(_L2["nchunk"], _L2["chrows"])
_ROW1 = _make_row_kernel(roww=128, tc_tiling=True, **_L1)
_ROW2 = _make_row_kernel(roww=40, **_L2)
_SCALE128 = _make_scale_kernel(128)
_SCALE40 = _make_scale_kernel(40)


def _cnt_assemble(cnt_raw, nchunk, chrows):
    histlen = _histlen(chrows)
    return cnt_raw.reshape(nchunk, histlen)[:, :chrows].reshape(NP, 1)


def kernel(x, hyperedges, W1, W2, rv1, rv2, bn_gamma, bn_beta):
    heT = jnp.pad(hyperedges, ((0, EHP - EH), (0, 0))).T.reshape(-1)
    ones2d = jnp.ones((EPT // 128, 128), jnp.float32)
    zerosd = jnp.zeros((DHIST,), jnp.float32)
    g2d = (bn_gamma / jnp.sqrt(1.0 + 1e-5)).reshape(1, -1)
    b2d = bn_beta.reshape(1, -1)

    X1, proj1 = _matmul_proj(x, W1, rv1)
    bd1, bs1, nb1, cntr1 = _PREP1(heT, proj1.reshape(-1), ones2d, zerosd)
    Y1, dis1 = _SCALE128(X1, cntr1[:NP, None], cntr1[DHIST:DHIST + NP, None])
    zp1 = _ROW1(Y1, bd1, bs1, nb1)

    X2, proj2 = _bn_mm(zp1, dis1, g2d, b2d, W2, rv2)
    bd2, bs2, nb2, cntr2 = _PREP2(heT, proj2.reshape(-1), ones2d, zerosd)
    Y2, dis2 = _SCALE40(X2, cntr2[:NP, None], cntr2[DHIST:DHIST + NP, None])
    zp2 = _ROW2(Y2, bd2, bs2, nb2)

    return _lsm(zp2, dis2, W2.shape[1])


# speedup vs baseline: 1.0309x; 1.0309x over previous
import functools

import jax
import jax.numpy as jnp
from jax import lax
from jax.experimental import pallas as pl
from jax.experimental.pallas import tpu as pltpu
from jax.experimental.pallas import tpu_sc as plsc

N = 100000
NP = 102400
EH = 100000
EHP = 102400
EPT = EHP // 32
G = 96
GPAD = 128
CAPB = 6400
DUMP = NP
_SC_MESH = dict(core_axis_name="c", subcore_axis_name="s")


def _scalar_lane(vec, i):
    return jnp.sum(jnp.where(lax.iota(jnp.int32, 16) == i, vec, jnp.int32(0)))


def _uv_body(heT, proj_hbm, u_hbm, v_hbm, proj_v, e_vs, u_v, v_v):
    wid = lax.axis_index("c") * 16 + lax.axis_index("s")
    base = wid * EPT
    pltpu.sync_copy(proj_hbm.at[pl.ds(0, N)], proj_v)
    for k in range(4):
        pltpu.sync_copy(heT.at[pl.ds(k * EHP + base, EPT)], e_vs[k])

    def body(i, carry):
        off = i * 16
        e0 = e_vs[0][pl.ds(off, 16)]
        p0 = plsc.load_gather(proj_v, [e0])
        ubest, pmax = e0, p0
        vbest, pmin = e0, p0
        for k in range(1, 4):
            ek = e_vs[k][pl.ds(off, 16)]
            pk = plsc.load_gather(proj_v, [ek])
            mx = pk > pmax
            ubest = jnp.where(mx, ek, ubest)
            pmax = jnp.where(mx, pk, pmax)
            mn = pk < pmin
            vbest = jnp.where(mn, ek, vbest)
            pmin = jnp.where(mn, pk, pmin)
        valid = (base + off + lax.iota(jnp.int32, 16)) < EH
        dumpv = DUMP + lax.iota(jnp.int32, 16)
        u_v[pl.ds(off, 16)] = jnp.where(valid, ubest, dumpv)
        v_v[pl.ds(off, 16)] = jnp.where(valid, vbest, dumpv)
        return carry

    lax.fori_loop(0, EPT // 16, body, 0)
    pltpu.sync_copy(u_v, u_hbm.at[pl.ds(base, EPT)])
    pltpu.sync_copy(v_v, v_hbm.at[pl.ds(base, EPT)])


@functools.partial(
    pl.kernel,
    out_type=[
        jax.ShapeDtypeStruct((EHP,), jnp.int32),
        jax.ShapeDtypeStruct((EHP,), jnp.int32),
    ],
    mesh=plsc.VectorSubcoreMesh(**_SC_MESH),
    scratch_types=[
        pltpu.VMEM((N,), jnp.float32),
        [pltpu.VMEM((EPT,), jnp.int32)] * 4,
        pltpu.VMEM((EPT,), jnp.int32),
        pltpu.VMEM((EPT,), jnp.int32),
    ],
    compiler_params=pltpu.CompilerParams(needs_layout_passes=False, use_tc_tiling_on_sc=False),
)
def _uv_kernel(heT, proj, u_out, v_out, proj_v, e_vs, u_v, v_v):
    _uv_body(heT, proj, u_out, v_out, proj_v, e_vs, u_v, v_v)


def _make_bin_kernel(nchunk, chrows):
    def body(u_hbm, v_hbm, bd_hbm, bs_hbm, nb_hbm, u_v, v_v, bd_vs, bs_vs, nb_v):
        wid = lax.axis_index("c") * 16 + lax.axis_index("s")
        base = wid * EPT
        pltpu.sync_copy(u_hbm.at[pl.ds(base, EPT)], u_v)
        pltpu.sync_copy(v_hbm.at[pl.ds(base, EPT)], v_v)
        iot = lax.iota(jnp.int32, 16)

        def it(i, cnts):
            off = i * 16
            uu = u_v[pl.ds(off, 16)]
            vv = v_v[pl.ds(off, 16)]
            cnts = list(cnts)
            for dd, ss in ((vv, uu), (uu, vv)):
                for c in range(nchunk):
                    lo = c * chrows
                    m = (dd >= lo) & (dd < lo + chrows)
                    mi = m.astype(jnp.int32)
                    incl = plsc.cumsum(mi)
                    pos = cnts[c] + incl - mi
                    plsc.store_scatter(bd_vs[c], [pos], dd - lo, mask=m)
                    plsc.store_scatter(bs_vs[c], [pos], ss, mask=m)
                    cnts[c] = cnts[c] + jnp.max(incl)
            return tuple(cnts)

        cnts = lax.fori_loop(0, EPT // 16, it, (jnp.int32(0),) * nchunk)
        nbvec = jnp.zeros((16,), jnp.int32)
        for c in range(nchunk):
            for k in range(GPAD // 16):
                pos = cnts[c] + k * 16 + iot
                plsc.store_scatter(bd_vs[c], [pos], chrows + iot)
                plsc.store_scatter(bs_vs[c], [pos], wid * GPAD + k * 16 + iot)
            nbvec = jnp.where(iot == c, cnts[c], nbvec)
        nb_v[pl.ds(0, 16)] = nbvec
        pltpu.sync_copy(nb_v, nb_hbm.at[pl.ds(wid * 16, 16)])
        for c in range(nchunk):
            pltpu.sync_copy(bd_vs[c], bd_hbm.at[pl.ds((c * 32 + wid) * CAPB, CAPB)])
            pltpu.sync_copy(bs_vs[c], bs_hbm.at[pl.ds((c * 32 + wid) * CAPB, CAPB)])

    return pl.kernel(
        body,
        out_type=[
            jax.ShapeDtypeStruct((nchunk * 32 * CAPB,), jnp.int32),
            jax.ShapeDtypeStruct((nchunk * 32 * CAPB,), jnp.int32),
            jax.ShapeDtypeStruct((512,), jnp.int32),
        ],
        mesh=plsc.VectorSubcoreMesh(**_SC_MESH),
        scratch_types=[
            pltpu.VMEM((EPT,), jnp.int32),
            pltpu.VMEM((EPT,), jnp.int32),
            [pltpu.VMEM((CAPB,), jnp.int32)] * nchunk,
            [pltpu.VMEM((CAPB,), jnp.int32)] * nchunk,
            pltpu.VMEM((16,), jnp.int32),
        ],
        compiler_params=pltpu.CompilerParams(needs_layout_passes=False, use_tc_tiling_on_sc=False),
    )


def _histlen(chrows):
    return -(-(chrows + 16) // 256) * 256


DHIST = 102656
_DSTRIPE = DHIST // 16


def _deg_body(u2_hbm, v2_hbm, ones_hbm, zeros_hbm, cnt_hbm, hist_sh, ub2, vb2, ones_b, sem):
    cid = lax.axis_index("c")
    sid = lax.axis_index("s")
    wid = cid * 16 + sid
    rows = EPT // 128
    pltpu.sync_copy(zeros_hbm.at[pl.ds(sid * _DSTRIPE, _DSTRIPE)],
                    hist_sh.at[pl.ds(sid * _DSTRIPE, _DSTRIPE)])
    pltpu.sync_copy(ones_hbm, ones_b)
    pltpu.sync_copy(u2_hbm.at[pl.ds(wid * rows, rows)], ub2)
    pltpu.sync_copy(v2_hbm.at[pl.ds(wid * rows, rows)], vb2)
    plsc.subcore_barrier()
    hs = []
    for b in range(rows):
        hs.append(pltpu.async_copy(ones_b.at[b], hist_sh.at[ub2.at[b]], sem, add=True))
        hs.append(pltpu.async_copy(ones_b.at[b], hist_sh.at[vb2.at[b]], sem, add=True))
    for h in hs:
        h.wait()
    plsc.subcore_barrier()
    pltpu.sync_copy(hist_sh.at[pl.ds(sid * _DSTRIPE, _DSTRIPE)],
                    cnt_hbm.at[pl.ds(cid * DHIST + sid * _DSTRIPE, _DSTRIPE)])


_deg_kernel = pl.kernel(
    _deg_body,
    out_type=jax.ShapeDtypeStruct((2 * DHIST,), jnp.float32),
    mesh=plsc.VectorSubcoreMesh(**_SC_MESH),
    scratch_types=[
        pltpu.VMEM_SHARED((DHIST,), jnp.float32),
        pltpu.VMEM((EPT // 128, 128), jnp.int32),
        pltpu.VMEM((EPT // 128, 128), jnp.int32),
        pltpu.VMEM((EPT // 128, 128), jnp.float32),
        pltpu.SemaphoreType.DMA,
    ],
    compiler_params=pltpu.CompilerParams(needs_layout_passes=False, use_tc_tiling_on_sc=False),
)


def _make_prep_kernel(nchunk, chrows):
    rows = EPT // 128

    def body(heT, proj_hbm, ones_hbm, zeros_hbm,
             bd_hbm, bs_hbm, nb_hbm, cnt_hbm,
             u2, v2, nb_v, hist_sh, ones_b, sem):
        cid = lax.axis_index("c")
        sid = lax.axis_index("s")
        wid = cid * 16 + sid
        base = wid * EPT
        iot = lax.iota(jnp.int32, 16)

        def phase_a(proj_v, e_vs):
            pltpu.sync_copy(proj_hbm.at[pl.ds(0, N)], proj_v)
            for k in range(4):
                pltpu.sync_copy(heT.at[pl.ds(k * EHP + base, EPT)], e_vs[k])

            def rowit(r, carry):
                for k in range(8):
                    off = r * 128 + k * 16
                    e0 = e_vs[0][pl.ds(off, 16)]
                    p0 = plsc.load_gather(proj_v, [e0])
                    ubest, pmax = e0, p0
                    vbest, pmin = e0, p0
                    for q in range(1, 4):
                        eq = e_vs[q][pl.ds(off, 16)]
                        pq = plsc.load_gather(proj_v, [eq])
                        mx = pq > pmax
                        ubest = jnp.where(mx, eq, ubest)
                        pmax = jnp.where(mx, pq, pmax)
                        mn = pq < pmin
                        vbest = jnp.where(mn, eq, vbest)
                        pmin = jnp.where(mn, pq, pmin)
                    valid = (base + off + iot) < EH
                    dumpv = DUMP + iot
                    u2[r, pl.ds(k * 16, 16)] = jnp.where(valid, ubest, dumpv)
                    v2[r, pl.ds(k * 16, 16)] = jnp.where(valid, vbest, dumpv)
                return carry

            lax.fori_loop(0, rows, rowit, 0)

        pl.run_scoped(
            phase_a,
            pltpu.VMEM((N,), jnp.float32),
            [pltpu.VMEM((EPT,), jnp.int32)] * 4,
        )

        def phase_b(bd_vs, bs_vs):
            def rowit(r, cnts):
                cnts = list(cnts)
                for k in range(8):
                    uu = u2[r, pl.ds(k * 16, 16)]
                    vv = v2[r, pl.ds(k * 16, 16)]
                    for dd, ss in ((vv, uu), (uu, vv)):
                        for c in range(nchunk):
                            lo = c * chrows
                            m = (dd >= lo) & (dd < lo + chrows)
                            mi = m.astype(jnp.int32)
                            incl = plsc.cumsum(mi)
                            pos = cnts[c] + incl - mi
                            plsc.store_scatter(bd_vs[c], [pos], dd - lo, mask=m)
                            plsc.store_scatter(bs_vs[c], [pos], ss, mask=m)
                            cnts[c] = cnts[c] + jnp.max(incl)
                return tuple(cnts)

            cnts = lax.fori_loop(0, rows, rowit, (jnp.int32(0),) * nchunk)
            nbvec = jnp.zeros((16,), jnp.int32)
            for c in range(nchunk):
                for k in range(GPAD // 16):
                    pos = cnts[c] + k * 16 + iot
                    plsc.store_scatter(bd_vs[c], [pos], chrows + iot)
                    plsc.store_scatter(bs_vs[c], [pos], wid * GPAD + k * 16 + iot)
                nbvec = jnp.where(iot == c, cnts[c], nbvec)
            nb_v[pl.ds(0, 16)] = nbvec
            pltpu.sync_copy(nb_v, nb_hbm.at[pl.ds(wid * 16, 16)])
            for c in range(nchunk):
                pltpu.sync_copy(bd_vs[c], bd_hbm.at[pl.ds((c * 32 + wid) * CAPB, CAPB)])
                pltpu.sync_copy(bs_vs[c], bs_hbm.at[pl.ds((c * 32 + wid) * CAPB, CAPB)])

        pl.run_scoped(
            phase_b,
            [pltpu.VMEM((CAPB,), jnp.int32)] * nchunk,
            [pltpu.VMEM((CAPB,), jnp.int32)] * nchunk,
        )

        pltpu.sync_copy(zeros_hbm.at[pl.ds(sid * _DSTRIPE, _DSTRIPE)],
                        hist_sh.at[pl.ds(sid * _DSTRIPE, _DSTRIPE)])
        pltpu.sync_copy(ones_hbm, ones_b)
        plsc.subcore_barrier()
        hs = []
        for b in range(rows):
            hs.append(pltpu.async_copy(ones_b.at[b], hist_sh.at[u2.at[b]], sem, add=True))
            hs.append(pltpu.async_copy(ones_b.at[b], hist_sh.at[v2.at[b]], sem, add=True))
        for h in hs:
            h.wait()
        plsc.subcore_barrier()
        pltpu.sync_copy(hist_sh.at[pl.ds(sid * _DSTRIPE, _DSTRIPE)],
                        cnt_hbm.at[pl.ds(cid * DHIST + sid * _DSTRIPE, _DSTRIPE)])

    return pl.kernel(
        body,
        out_type=[
            jax.ShapeDtypeStruct((nchunk * 32 * CAPB,), jnp.int32),
            jax.ShapeDtypeStruct((nchunk * 32 * CAPB,), jnp.int32),
            jax.ShapeDtypeStruct((512,), jnp.int32),
            jax.ShapeDtypeStruct((2 * DHIST,), jnp.float32),
        ],
        mesh=plsc.VectorSubcoreMesh(**_SC_MESH),
        scratch_types=[
            pltpu.VMEM((EPT // 128, 128), jnp.int32),
            pltpu.VMEM((EPT // 128, 128), jnp.int32),
            pltpu.VMEM((16,), jnp.int32),
            pltpu.VMEM_SHARED((DHIST,), jnp.float32),
            pltpu.VMEM((EPT // 128, 128), jnp.float32),
            pltpu.SemaphoreType.DMA,
        ],
        compiler_params=pltpu.CompilerParams(needs_layout_passes=False, use_tc_tiling_on_sc=False),
    )


def _make_row_kernel(nchunk, chrows, cpc, roww, tc_tiling=False, g=G):
    accr = chrows + 16
    sr = chrows // 16

    def body(y_hbm, bd_hbm, bs_hbm, nb_hbm, zp_hbm, acc_sh,
             db3, sb3, rows3, cbuf, sg):
        cid = lax.axis_index("c")
        sid = lax.axis_index("s")

        def seg_loop(c, w):
            pltpu.sync_copy(nb_hbm.at[pl.ds(w * 16, 16)], cbuf)
            cnt = _scalar_lane(cbuf[pl.ds(0, 16)], c)
            seg = (c * 32 + w) * CAPB

            def step(b, carry):
                par = lax.rem(b, 2)

                @pl.when(b * g < cnt)
                def _():
                    pltpu.sync_copy(bd_hbm.at[pl.ds(seg + b * g, g)], db3.at[par])
                    pltpu.sync_copy(bs_hbm.at[pl.ds(seg + b * g, g)], sb3.at[par])
                    pltpu.async_copy(y_hbm.at[sb3.at[par]], rows3.at[par], sg)

                @pl.when((b > 0) & ((b - 1) * g < cnt))
                def _():
                    q = 1 - par
                    pltpu.make_async_copy(y_hbm.at[sb3.at[q]], rows3.at[q], sg).wait()
                    pltpu.sync_copy(rows3.at[q], acc_sh.at[db3.at[q]], add=True)

                return carry

            lax.fori_loop(0, lax.shift_right_logical(cnt, 6) + 2, step, 0)

        for kk in range(cpc):
            c = cid * cpc + kk
            pltpu.sync_copy(y_hbm.at[pl.ds(c * chrows + sid * sr, sr)],
                            acc_sh.at[pl.ds(sid * sr, sr)])
            plsc.subcore_barrier()
            for j in range(2):
                seg_loop(c, 2 * sid + j)
            plsc.subcore_barrier()
            pltpu.sync_copy(acc_sh.at[pl.ds(sid * sr, sr)],
                            zp_hbm.at[pl.ds(c * chrows + sid * sr, sr)])

    return pl.kernel(
        body,
        out_type=jax.ShapeDtypeStruct((NP, roww), jnp.float32),
        mesh=plsc.VectorSubcoreMesh(**_SC_MESH),
        scratch_types=[
            pltpu.VMEM_SHARED((accr, roww), jnp.float32),
            pltpu.VMEM((2, g), jnp.int32),
            pltpu.VMEM((2, g), jnp.int32),
            pltpu.VMEM((2, g, roww), jnp.float32),
            pltpu.VMEM((16,), jnp.int32),
            pltpu.SemaphoreType.DMA,
        ],
        compiler_params=pltpu.CompilerParams(
            needs_layout_passes=False, use_tc_tiling_on_sc=tc_tiling),
    )


def _mm_proj_body(x_ref, W_ref, rv_ref, X_ref, p_ref):
    X = jnp.dot(x_ref[...], W_ref[...], preferred_element_type=jnp.float32)
    X_ref[...] = X
    p_ref[...] = jnp.dot(X, rv_ref[...], preferred_element_type=jnp.float32)


def _matmul_proj(x, W, rv, Bn=800):
    n, d = x.shape
    h = W.shape[1]
    return pl.pallas_call(
        _mm_proj_body,
        grid=(n // Bn,),
        in_specs=[
            pl.BlockSpec((Bn, d), lambda i: (i, 0)),
            pl.BlockSpec((d, h), lambda i: (0, 0)),
            pl.BlockSpec((h, 1), lambda i: (0, 0)),
        ],
        out_specs=[
            pl.BlockSpec((Bn, h), lambda i: (i, 0)),
            pl.BlockSpec((Bn, 1), lambda i: (i, 0)),
        ],
        out_shape=[
            jax.ShapeDtypeStruct((NP, h), jnp.float32),
            jax.ShapeDtypeStruct((NP, 1), jnp.float32),
        ],
    )(x, W, rv)


def _scale_body(x_ref, ca_ref, cb_ref, y_ref, dis_ref):
    dis = lax.rsqrt(jnp.maximum(ca_ref[...] + cb_ref[...] + 1.0, 1e-12))
    dis_ref[...] = dis
    y_ref[...] = x_ref[...] * dis


def _mm_scale_body(x_ref, W_ref, ca_ref, cb_ref, y_ref, dis_ref):
    dis = lax.rsqrt(jnp.maximum(ca_ref[...] + cb_ref[...] + 1.0, 1e-12))
    dis_ref[...] = dis
    X = jnp.dot(x_ref[...], W_ref[...], preferred_element_type=jnp.float32)
    y_ref[...] = X * dis


def _mm_scale(x, W, ca, cb, Bn=800):
    n, d = x.shape
    h = W.shape[1]
    return pl.pallas_call(
        _mm_scale_body,
        grid=(n // Bn,),
        in_specs=[
            pl.BlockSpec((Bn, d), lambda i: (i, 0)),
            pl.BlockSpec((d, h), lambda i: (0, 0)),
            pl.BlockSpec((Bn, 1), lambda i: (i, 0)),
            pl.BlockSpec((Bn, 1), lambda i: (i, 0)),
        ],
        out_specs=[
            pl.BlockSpec((Bn, h), lambda i: (i, 0)),
            pl.BlockSpec((Bn, 1), lambda i: (i, 0)),
        ],
        out_shape=[
            jax.ShapeDtypeStruct((NP, h), jnp.float32),
            jax.ShapeDtypeStruct((NP, 1), jnp.float32),
        ],
    )(x, W, ca, cb)


def _make_scale_kernel(roww, Bn=2048):
    return pl.pallas_call(
        _scale_body,
        grid=(NP // Bn,),
        in_specs=[
            pl.BlockSpec((Bn, roww), lambda i: (i, 0)),
            pl.BlockSpec((Bn, 1), lambda i: (i, 0)),
            pl.BlockSpec((Bn, 1), lambda i: (i, 0)),
        ],
        out_specs=[
            pl.BlockSpec((Bn, roww), lambda i: (i, 0)),
            pl.BlockSpec((Bn, 1), lambda i: (i, 0)),
        ],
        out_shape=[
            jax.ShapeDtypeStruct((NP, roww), jnp.float32),
            jax.ShapeDtypeStruct((NP, 1), jnp.float32),
        ],
    )


def _bn_mm_body(zp_ref, dis_ref, g_ref, b_ref, W_ref, rv_ref, x2_ref, p2_ref):
    t = jnp.maximum(zp_ref[...] * dis_ref[...], 0.0)
    t = t * g_ref[...] + b_ref[...]
    X2 = jnp.dot(t, W_ref[...], preferred_element_type=jnp.float32)
    x2_ref[...] = X2
    p2_ref[...] = jnp.dot(X2, rv_ref[...], preferred_element_type=jnp.float32)


def _bn_mm(zp, dis, g2d, b2d, W2, rv2, Bn=2048):
    h, c = W2.shape
    return pl.pallas_call(
        _bn_mm_body,
        grid=(NP // Bn,),
        in_specs=[
            pl.BlockSpec((Bn, h), lambda i: (i, 0)),
            pl.BlockSpec((Bn, 1), lambda i: (i, 0)),
            pl.BlockSpec((1, h), lambda i: (0, 0)),
            pl.BlockSpec((1, h), lambda i: (0, 0)),
            pl.BlockSpec((h, c), lambda i: (0, 0)),
            pl.BlockSpec((c, 1), lambda i: (0, 0)),
        ],
        out_specs=[
            pl.BlockSpec((Bn, c), lambda i: (i, 0)),
            pl.BlockSpec((Bn, 1), lambda i: (i, 0)),
        ],
        out_shape=[
            jax.ShapeDtypeStruct((NP, c), jnp.float32),
            jax.ShapeDtypeStruct((NP, 1), jnp.float32),
        ],
    )(zp, dis, g2d, b2d, W2, rv2)


def _lsm_body(zp_ref, dis_ref, out_ref):
    L = zp_ref[...] * dis_ref[...]
    m = jnp.max(L, axis=-1, keepdims=True)
    s = jnp.log(jnp.sum(jnp.exp(L - m), axis=-1, keepdims=True))
    out_ref[...] = L - m - s


def _lsm(zp, dis, c, Bn=800):
    return pl.pallas_call(
        _lsm_body,
        grid=(N // Bn,),
        in_specs=[
            pl.BlockSpec((Bn, c), lambda i: (i, 0)),
            pl.BlockSpec((Bn, 1), lambda i: (i, 0)),
        ],
        out_specs=pl.BlockSpec((Bn, c), lambda i: (i, 0)),
        out_shape=jax.ShapeDtypeStruct((N, c), jnp.float32),
    )(zp, dis)


_L1 = dict(nchunk=8, chrows=12800, cpc=4)
_L2 = dict(nchunk=4, chrows=25600, cpc=2)
_BIN1 = _make_bin_kernel(_L1["nchunk"], _L1["chrows"])
_BIN2 = _make_bin_kernel(_L2["nchunk"], _L2["chrows"])
_ROW1 = _make_row_kernel(roww=128, tc_tiling=True, **_L1)
_ROW2 = _make_row_kernel(roww=40, g=128, **_L2)
_SCALE128 = _make_scale_kernel(128)
_SCALE40 = _make_scale_kernel(40)


def _cnt_assemble(cnt_raw, nchunk, chrows):
    histlen = _histlen(chrows)
    return cnt_raw.reshape(nchunk, histlen)[:, :chrows].reshape(NP, 1)


def kernel(x, hyperedges, W1, W2, rv1, rv2, bn_gamma, bn_beta):
    heT = jnp.pad(hyperedges, ((0, EHP - EH), (0, 0))).T.reshape(-1)
    ones2d = jnp.ones((EPT // 128, 128), jnp.float32)
    zerosd = jnp.zeros((DHIST,), jnp.float32)
    g2d = (bn_gamma / jnp.sqrt(1.0 + 1e-5)).reshape(1, -1)
    b2d = bn_beta.reshape(1, -1)

    X1, proj1 = _matmul_proj(x, W1, rv1)
    u1, v1 = _uv_kernel(heT, proj1.reshape(-1))
    bd1, bs1, nb1 = _BIN1(u1, v1)
    cntr1 = _deg_kernel(u1.reshape(-1, 128), v1.reshape(-1, 128), ones2d, zerosd)
    Y1, dis1 = _SCALE128(X1, cntr1[:NP, None], cntr1[DHIST:DHIST + NP, None])
    zp1 = _ROW1(Y1, bd1, bs1, nb1)

    X2, proj2 = _bn_mm(zp1, dis1, g2d, b2d, W2, rv2)
    u2, v2 = _uv_kernel(heT, proj2.reshape(-1))
    bd2, bs2, nb2 = _BIN2(u2, v2)
    cntr2 = _deg_kernel(u2.reshape(-1, 128), v2.reshape(-1, 128), ones2d, zerosd)
    Y2, dis2 = _SCALE40(X2, cntr2[:NP, None], cntr2[DHIST:DHIST + NP, None])
    zp2 = _ROW2(Y2, bd2, bs2, nb2)

    return _lsm(zp2, dis2, W2.shape[1])


# speedup vs baseline: 1.0366x; 1.0055x over previous
import functools

import jax
import jax.numpy as jnp
from jax import lax
from jax.experimental import pallas as pl
from jax.experimental.pallas import tpu as pltpu
from jax.experimental.pallas import tpu_sc as plsc

N = 100000
NP = 102400
EH = 100000
EHP = 102400
EPT = EHP // 32
G = 96
GPAD = 128
CAPB = 6400
DUMP = NP
_SC_MESH = dict(core_axis_name="c", subcore_axis_name="s")


def _scalar_lane(vec, i):
    return jnp.sum(jnp.where(lax.iota(jnp.int32, 16) == i, vec, jnp.int32(0)))


def _uv_body(heT, proj_hbm, u_hbm, v_hbm, proj_v, e_vs, u_v, v_v):
    wid = lax.axis_index("c") * 16 + lax.axis_index("s")
    base = wid * EPT
    pltpu.sync_copy(proj_hbm.at[pl.ds(0, N)], proj_v)
    for k in range(4):
        pltpu.sync_copy(heT.at[pl.ds(k * EHP + base, EPT)], e_vs[k])

    def body(i, carry):
        off = i * 16
        e0 = e_vs[0][pl.ds(off, 16)]
        p0 = plsc.load_gather(proj_v, [e0])
        ubest, pmax = e0, p0
        vbest, pmin = e0, p0
        for k in range(1, 4):
            ek = e_vs[k][pl.ds(off, 16)]
            pk = plsc.load_gather(proj_v, [ek])
            mx = pk > pmax
            ubest = jnp.where(mx, ek, ubest)
            pmax = jnp.where(mx, pk, pmax)
            mn = pk < pmin
            vbest = jnp.where(mn, ek, vbest)
            pmin = jnp.where(mn, pk, pmin)
        valid = (base + off + lax.iota(jnp.int32, 16)) < EH
        dumpv = DUMP + lax.iota(jnp.int32, 16)
        u_v[pl.ds(off, 16)] = jnp.where(valid, ubest, dumpv)
        v_v[pl.ds(off, 16)] = jnp.where(valid, vbest, dumpv)
        return carry

    lax.fori_loop(0, EPT // 16, body, 0)
    pltpu.sync_copy(u_v, u_hbm.at[pl.ds(base, EPT)])
    pltpu.sync_copy(v_v, v_hbm.at[pl.ds(base, EPT)])


@functools.partial(
    pl.kernel,
    out_type=[
        jax.ShapeDtypeStruct((EHP,), jnp.int32),
        jax.ShapeDtypeStruct((EHP,), jnp.int32),
    ],
    mesh=plsc.VectorSubcoreMesh(**_SC_MESH),
    scratch_types=[
        pltpu.VMEM((N,), jnp.float32),
        [pltpu.VMEM((EPT,), jnp.int32)] * 4,
        pltpu.VMEM((EPT,), jnp.int32),
        pltpu.VMEM((EPT,), jnp.int32),
    ],
    compiler_params=pltpu.CompilerParams(needs_layout_passes=False, use_tc_tiling_on_sc=False),
)
def _uv_kernel(heT, proj, u_out, v_out, proj_v, e_vs, u_v, v_v):
    _uv_body(heT, proj, u_out, v_out, proj_v, e_vs, u_v, v_v)


def _make_bin_kernel(nchunk, chrows):
    def body(u_hbm, v_hbm, bd_hbm, bs_hbm, nb_hbm, u_v, v_v, bd_vs, bs_vs, nb_v):
        wid = lax.axis_index("c") * 16 + lax.axis_index("s")
        base = wid * EPT
        pltpu.sync_copy(u_hbm.at[pl.ds(base, EPT)], u_v)
        pltpu.sync_copy(v_hbm.at[pl.ds(base, EPT)], v_v)
        iot = lax.iota(jnp.int32, 16)

        def it(i, cnts):
            off = i * 16
            uu = u_v[pl.ds(off, 16)]
            vv = v_v[pl.ds(off, 16)]
            cnts = list(cnts)
            for dd, ss in ((vv, uu), (uu, vv)):
                for c in range(nchunk):
                    lo = c * chrows
                    m = (dd >= lo) & (dd < lo + chrows)
                    mi = m.astype(jnp.int32)
                    incl = plsc.cumsum(mi)
                    pos = cnts[c] + incl - mi
                    plsc.store_scatter(bd_vs[c], [pos], dd - lo, mask=m)
                    plsc.store_scatter(bs_vs[c], [pos], ss, mask=m)
                    cnts[c] = cnts[c] + jnp.max(incl)
            return tuple(cnts)

        cnts = lax.fori_loop(0, EPT // 16, it, (jnp.int32(0),) * nchunk)
        nbvec = jnp.zeros((16,), jnp.int32)
        for c in range(nchunk):
            for k in range(GPAD // 16):
                pos = cnts[c] + k * 16 + iot
                plsc.store_scatter(bd_vs[c], [pos], chrows + iot)
                plsc.store_scatter(bs_vs[c], [pos], wid * GPAD + k * 16 + iot)
            nbvec = jnp.where(iot == c, cnts[c], nbvec)
        nb_v[pl.ds(0, 16)] = nbvec
        pltpu.sync_copy(nb_v, nb_hbm.at[pl.ds(wid * 16, 16)])
        for c in range(nchunk):
            pltpu.sync_copy(bd_vs[c], bd_hbm.at[pl.ds((c * 32 + wid) * CAPB, CAPB)])
            pltpu.sync_copy(bs_vs[c], bs_hbm.at[pl.ds((c * 32 + wid) * CAPB, CAPB)])

    return pl.kernel(
        body,
        out_type=[
            jax.ShapeDtypeStruct((nchunk * 32 * CAPB,), jnp.int32),
            jax.ShapeDtypeStruct((nchunk * 32 * CAPB,), jnp.int32),
            jax.ShapeDtypeStruct((512,), jnp.int32),
        ],
        mesh=plsc.VectorSubcoreMesh(**_SC_MESH),
        scratch_types=[
            pltpu.VMEM((EPT,), jnp.int32),
            pltpu.VMEM((EPT,), jnp.int32),
            [pltpu.VMEM((CAPB,), jnp.int32)] * nchunk,
            [pltpu.VMEM((CAPB,), jnp.int32)] * nchunk,
            pltpu.VMEM((16,), jnp.int32),
        ],
        compiler_params=pltpu.CompilerParams(needs_layout_passes=False, use_tc_tiling_on_sc=False),
    )


def _histlen(chrows):
    return -(-(chrows + 16) // 256) * 256


DHIST = 102656
_DSTRIPE = DHIST // 16


def _deg_body(u2_hbm, v2_hbm, ones_hbm, zeros_hbm, cnt_hbm, hist_sh, ub2, vb2, ones_b, sem):
    cid = lax.axis_index("c")
    sid = lax.axis_index("s")
    wid = cid * 16 + sid
    rows = EPT // 128
    pltpu.sync_copy(zeros_hbm.at[pl.ds(sid * _DSTRIPE, _DSTRIPE)],
                    hist_sh.at[pl.ds(sid * _DSTRIPE, _DSTRIPE)])
    pltpu.sync_copy(ones_hbm, ones_b)
    pltpu.sync_copy(u2_hbm.at[pl.ds(wid * rows, rows)], ub2)
    pltpu.sync_copy(v2_hbm.at[pl.ds(wid * rows, rows)], vb2)
    plsc.subcore_barrier()
    hs = []
    for b in range(rows):
        hs.append(pltpu.async_copy(ones_b.at[b], hist_sh.at[ub2.at[b]], sem, add=True))
        hs.append(pltpu.async_copy(ones_b.at[b], hist_sh.at[vb2.at[b]], sem, add=True))
    for h in hs:
        h.wait()
    plsc.subcore_barrier()
    pltpu.sync_copy(hist_sh.at[pl.ds(sid * _DSTRIPE, _DSTRIPE)],
                    cnt_hbm.at[pl.ds(cid * DHIST + sid * _DSTRIPE, _DSTRIPE)])


_deg_kernel = pl.kernel(
    _deg_body,
    out_type=jax.ShapeDtypeStruct((2 * DHIST,), jnp.float32),
    mesh=plsc.VectorSubcoreMesh(**_SC_MESH),
    scratch_types=[
        pltpu.VMEM_SHARED((DHIST,), jnp.float32),
        pltpu.VMEM((EPT // 128, 128), jnp.int32),
        pltpu.VMEM((EPT // 128, 128), jnp.int32),
        pltpu.VMEM((EPT // 128, 128), jnp.float32),
        pltpu.SemaphoreType.DMA,
    ],
    compiler_params=pltpu.CompilerParams(needs_layout_passes=False, use_tc_tiling_on_sc=False),
)


def _make_prep_kernel(nchunk, chrows):
    rows = EPT // 128

    def body(heT, proj_hbm, ones_hbm, zeros_hbm,
             bd_hbm, bs_hbm, nb_hbm, cnt_hbm,
             u2, v2, nb_v, hist_sh, ones_b, sem):
        cid = lax.axis_index("c")
        sid = lax.axis_index("s")
        wid = cid * 16 + sid
        base = wid * EPT
        iot = lax.iota(jnp.int32, 16)

        def phase_a(proj_v, e_vs):
            pltpu.sync_copy(proj_hbm.at[pl.ds(0, N)], proj_v)
            for k in range(4):
                pltpu.sync_copy(heT.at[pl.ds(k * EHP + base, EPT)], e_vs[k])

            def rowit(r, carry):
                for k in range(8):
                    off = r * 128 + k * 16
                    e0 = e_vs[0][pl.ds(off, 16)]
                    p0 = plsc.load_gather(proj_v, [e0])
                    ubest, pmax = e0, p0
                    vbest, pmin = e0, p0
                    for q in range(1, 4):
                        eq = e_vs[q][pl.ds(off, 16)]
                        pq = plsc.load_gather(proj_v, [eq])
                        mx = pq > pmax
                        ubest = jnp.where(mx, eq, ubest)
                        pmax = jnp.where(mx, pq, pmax)
                        mn = pq < pmin
                        vbest = jnp.where(mn, eq, vbest)
                        pmin = jnp.where(mn, pq, pmin)
                    valid = (base + off + iot) < EH
                    dumpv = DUMP + iot
                    u2[r, pl.ds(k * 16, 16)] = jnp.where(valid, ubest, dumpv)
                    v2[r, pl.ds(k * 16, 16)] = jnp.where(valid, vbest, dumpv)
                return carry

            lax.fori_loop(0, rows, rowit, 0)

        pl.run_scoped(
            phase_a,
            pltpu.VMEM((N,), jnp.float32),
            [pltpu.VMEM((EPT,), jnp.int32)] * 4,
        )

        def phase_b(bd_vs, bs_vs):
            def rowit(r, cnts):
                cnts = list(cnts)
                for k in range(8):
                    uu = u2[r, pl.ds(k * 16, 16)]
                    vv = v2[r, pl.ds(k * 16, 16)]
                    for dd, ss in ((vv, uu), (uu, vv)):
                        for c in range(nchunk):
                            lo = c * chrows
                            m = (dd >= lo) & (dd < lo + chrows)
                            mi = m.astype(jnp.int32)
                            incl = plsc.cumsum(mi)
                            pos = cnts[c] + incl - mi
                            plsc.store_scatter(bd_vs[c], [pos], dd - lo, mask=m)
                            plsc.store_scatter(bs_vs[c], [pos], ss, mask=m)
                            cnts[c] = cnts[c] + jnp.max(incl)
                return tuple(cnts)

            cnts = lax.fori_loop(0, rows, rowit, (jnp.int32(0),) * nchunk)
            nbvec = jnp.zeros((16,), jnp.int32)
            for c in range(nchunk):
                for k in range(GPAD // 16):
                    pos = cnts[c] + k * 16 + iot
                    plsc.store_scatter(bd_vs[c], [pos], chrows + iot)
                    plsc.store_scatter(bs_vs[c], [pos], wid * GPAD + k * 16 + iot)
                nbvec = jnp.where(iot == c, cnts[c], nbvec)
            nb_v[pl.ds(0, 16)] = nbvec
            pltpu.sync_copy(nb_v, nb_hbm.at[pl.ds(wid * 16, 16)])
            for c in range(nchunk):
                pltpu.sync_copy(bd_vs[c], bd_hbm.at[pl.ds((c * 32 + wid) * CAPB, CAPB)])
                pltpu.sync_copy(bs_vs[c], bs_hbm.at[pl.ds((c * 32 + wid) * CAPB, CAPB)])

        pl.run_scoped(
            phase_b,
            [pltpu.VMEM((CAPB,), jnp.int32)] * nchunk,
            [pltpu.VMEM((CAPB,), jnp.int32)] * nchunk,
        )

        pltpu.sync_copy(zeros_hbm.at[pl.ds(sid * _DSTRIPE, _DSTRIPE)],
                        hist_sh.at[pl.ds(sid * _DSTRIPE, _DSTRIPE)])
        pltpu.sync_copy(ones_hbm, ones_b)
        plsc.subcore_barrier()
        hs = []
        for b in range(rows):
            hs.append(pltpu.async_copy(ones_b.at[b], hist_sh.at[u2.at[b]], sem, add=True))
            hs.append(pltpu.async_copy(ones_b.at[b], hist_sh.at[v2.at[b]], sem, add=True))
        for h in hs:
            h.wait()
        plsc.subcore_barrier()
        pltpu.sync_copy(hist_sh.at[pl.ds(sid * _DSTRIPE, _DSTRIPE)],
                        cnt_hbm.at[pl.ds(cid * DHIST + sid * _DSTRIPE, _DSTRIPE)])

    return pl.kernel(
        body,
        out_type=[
            jax.ShapeDtypeStruct((nchunk * 32 * CAPB,), jnp.int32),
            jax.ShapeDtypeStruct((nchunk * 32 * CAPB,), jnp.int32),
            jax.ShapeDtypeStruct((512,), jnp.int32),
            jax.ShapeDtypeStruct((2 * DHIST,), jnp.float32),
        ],
        mesh=plsc.VectorSubcoreMesh(**_SC_MESH),
        scratch_types=[
            pltpu.VMEM((EPT // 128, 128), jnp.int32),
            pltpu.VMEM((EPT // 128, 128), jnp.int32),
            pltpu.VMEM((16,), jnp.int32),
            pltpu.VMEM_SHARED((DHIST,), jnp.float32),
            pltpu.VMEM((EPT // 128, 128), jnp.float32),
            pltpu.SemaphoreType.DMA,
        ],
        compiler_params=pltpu.CompilerParams(needs_layout_passes=False, use_tc_tiling_on_sc=False),
    )


def _make_row_kernel(nchunk, chrows, cpc, roww, tc_tiling=False, g=G):
    accr = chrows + 16
    sr = chrows // 16

    def body(y_hbm, bd_hbm, bs_hbm, nb_hbm, zp_hbm, acc_sh,
             db3, sb3, rows3, cbuf, sg):
        cid = lax.axis_index("c")
        sid = lax.axis_index("s")

        def seg_loop(c, w):
            pltpu.sync_copy(nb_hbm.at[pl.ds(w * 16, 16)], cbuf)
            cnt = _scalar_lane(cbuf[pl.ds(0, 16)], c)
            seg = (c * 32 + w) * CAPB

            def step(b, carry):
                par = lax.rem(b, 2)

                @pl.when(b * g < cnt)
                def _():
                    pltpu.sync_copy(bd_hbm.at[pl.ds(seg + b * g, g)], db3.at[par])
                    pltpu.sync_copy(bs_hbm.at[pl.ds(seg + b * g, g)], sb3.at[par])
                    pltpu.async_copy(y_hbm.at[sb3.at[par]], rows3.at[par], sg)

                @pl.when((b > 0) & ((b - 1) * g < cnt))
                def _():
                    q = 1 - par
                    pltpu.make_async_copy(y_hbm.at[sb3.at[q]], rows3.at[q], sg).wait()
                    pltpu.sync_copy(rows3.at[q], acc_sh.at[db3.at[q]], add=True)

                return carry

            lax.fori_loop(0, lax.shift_right_logical(cnt, 6) + 2, step, 0)

        for kk in range(cpc):
            c = cid * cpc + kk
            pltpu.sync_copy(y_hbm.at[pl.ds(c * chrows + sid * sr, sr)],
                            acc_sh.at[pl.ds(sid * sr, sr)])
            plsc.subcore_barrier()
            for j in range(2):
                seg_loop(c, 2 * sid + j)
            plsc.subcore_barrier()
            pltpu.sync_copy(acc_sh.at[pl.ds(sid * sr, sr)],
                            zp_hbm.at[pl.ds(c * chrows + sid * sr, sr)])

    return pl.kernel(
        body,
        out_type=jax.ShapeDtypeStruct((NP, roww), jnp.float32),
        mesh=plsc.VectorSubcoreMesh(**_SC_MESH),
        scratch_types=[
            pltpu.VMEM_SHARED((accr, roww), jnp.float32),
            pltpu.VMEM((2, g), jnp.int32),
            pltpu.VMEM((2, g), jnp.int32),
            pltpu.VMEM((2, g, roww), jnp.float32),
            pltpu.VMEM((16,), jnp.int32),
            pltpu.SemaphoreType.DMA,
        ],
        compiler_params=pltpu.CompilerParams(
            needs_layout_passes=False, use_tc_tiling_on_sc=tc_tiling),
    )


def _mm_proj_body(x_ref, W_ref, rv_ref, X_ref, p_ref):
    X = jnp.dot(x_ref[...], W_ref[...], preferred_element_type=jnp.float32)
    X_ref[...] = X
    p_ref[...] = jnp.dot(X, rv_ref[...], preferred_element_type=jnp.float32)


def _matmul_proj(x, W, rv, Bn=800):
    n, d = x.shape
    h = W.shape[1]
    return pl.pallas_call(
        _mm_proj_body,
        grid=(n // Bn,),
        in_specs=[
            pl.BlockSpec((Bn, d), lambda i: (i, 0)),
            pl.BlockSpec((d, h), lambda i: (0, 0)),
            pl.BlockSpec((h, 1), lambda i: (0, 0)),
        ],
        out_specs=[
            pl.BlockSpec((Bn, h), lambda i: (i, 0)),
            pl.BlockSpec((Bn, 1), lambda i: (i, 0)),
        ],
        out_shape=[
            jax.ShapeDtypeStruct((NP, h), jnp.float32),
            jax.ShapeDtypeStruct((NP, 1), jnp.float32),
        ],
    )(x, W, rv)


def _scale_body(x_ref, ca_ref, cb_ref, y_ref, dis_ref):
    dis = lax.rsqrt(jnp.maximum(ca_ref[...] + cb_ref[...] + 1.0, 1e-12))
    dis_ref[...] = dis
    y_ref[...] = x_ref[...] * dis


def _mm_scale_body(x_ref, W_ref, ca_ref, cb_ref, y_ref, dis_ref):
    dis = lax.rsqrt(jnp.maximum(ca_ref[...] + cb_ref[...] + 1.0, 1e-12))
    dis_ref[...] = dis
    X = jnp.dot(x_ref[...], W_ref[...], preferred_element_type=jnp.float32)
    y_ref[...] = X * dis


def _mm_scale(x, W, ca, cb, Bn=800):
    n, d = x.shape
    h = W.shape[1]
    return pl.pallas_call(
        _mm_scale_body,
        grid=(n // Bn,),
        in_specs=[
            pl.BlockSpec((Bn, d), lambda i: (i, 0)),
            pl.BlockSpec((d, h), lambda i: (0, 0)),
            pl.BlockSpec((Bn, 1), lambda i: (i, 0)),
            pl.BlockSpec((Bn, 1), lambda i: (i, 0)),
        ],
        out_specs=[
            pl.BlockSpec((Bn, h), lambda i: (i, 0)),
            pl.BlockSpec((Bn, 1), lambda i: (i, 0)),
        ],
        out_shape=[
            jax.ShapeDtypeStruct((NP, h), jnp.float32),
            jax.ShapeDtypeStruct((NP, 1), jnp.float32),
        ],
    )(x, W, ca, cb)


def _make_scale_kernel(roww, Bn=2048):
    return pl.pallas_call(
        _scale_body,
        grid=(NP // Bn,),
        in_specs=[
            pl.BlockSpec((Bn, roww), lambda i: (i, 0)),
            pl.BlockSpec((Bn, 1), lambda i: (i, 0)),
            pl.BlockSpec((Bn, 1), lambda i: (i, 0)),
        ],
        out_specs=[
            pl.BlockSpec((Bn, roww), lambda i: (i, 0)),
            pl.BlockSpec((Bn, 1), lambda i: (i, 0)),
        ],
        out_shape=[
            jax.ShapeDtypeStruct((NP, roww), jnp.float32),
            jax.ShapeDtypeStruct((NP, 1), jnp.float32),
        ],
    )


def _bn_mm_body(zp_ref, dis_ref, g_ref, b_ref, W_ref, rv_ref, x2_ref, p2_ref):
    t = jnp.maximum(zp_ref[...] * dis_ref[...], 0.0)
    t = t * g_ref[...] + b_ref[...]
    X2 = jnp.dot(t, W_ref[...], preferred_element_type=jnp.float32)
    x2_ref[...] = X2
    p2_ref[...] = jnp.dot(X2, rv_ref[...], preferred_element_type=jnp.float32)


def _bn_mm(zp, dis, g2d, b2d, W2, rv2, Bn=2048):
    h, c = W2.shape
    return pl.pallas_call(
        _bn_mm_body,
        grid=(NP // Bn,),
        in_specs=[
            pl.BlockSpec((Bn, h), lambda i: (i, 0)),
            pl.BlockSpec((Bn, 1), lambda i: (i, 0)),
            pl.BlockSpec((1, h), lambda i: (0, 0)),
            pl.BlockSpec((1, h), lambda i: (0, 0)),
            pl.BlockSpec((h, c), lambda i: (0, 0)),
            pl.BlockSpec((c, 1), lambda i: (0, 0)),
        ],
        out_specs=[
            pl.BlockSpec((Bn, c), lambda i: (i, 0)),
            pl.BlockSpec((Bn, 1), lambda i: (i, 0)),
        ],
        out_shape=[
            jax.ShapeDtypeStruct((NP, c), jnp.float32),
            jax.ShapeDtypeStruct((NP, 1), jnp.float32),
        ],
    )(zp, dis, g2d, b2d, W2, rv2)


def _lsm_body(zp_ref, dis_ref, out_ref):
    L = zp_ref[...] * dis_ref[...]
    m = jnp.max(L, axis=-1, keepdims=True)
    s = jnp.log(jnp.sum(jnp.exp(L - m), axis=-1, keepdims=True))
    out_ref[...] = L - m - s


def _lsm(zp, dis, c, Bn=800):
    return pl.pallas_call(
        _lsm_body,
        grid=(N // Bn,),
        in_specs=[
            pl.BlockSpec((Bn, c), lambda i: (i, 0)),
            pl.BlockSpec((Bn, 1), lambda i: (i, 0)),
        ],
        out_specs=pl.BlockSpec((Bn, c), lambda i: (i, 0)),
        out_shape=jax.ShapeDtypeStruct((N, c), jnp.float32),
    )(zp, dis)


_L1 = dict(nchunk=8, chrows=12704, cpc=4)
_L2 = dict(nchunk=4, chrows=25600, cpc=2)
_BIN1 = _make_bin_kernel(_L1["nchunk"], _L1["chrows"])
_BIN2 = _make_bin_kernel(_L2["nchunk"], _L2["chrows"])
_ROW1 = _make_row_kernel(roww=128, g=112, **_L1)
_ROW2 = _make_row_kernel(roww=40, g=128, **_L2)
_SCALE128 = _make_scale_kernel(128)
_SCALE40 = _make_scale_kernel(40)


def _cnt_assemble(cnt_raw, nchunk, chrows):
    histlen = _histlen(chrows)
    return cnt_raw.reshape(nchunk, histlen)[:, :chrows].reshape(NP, 1)


def kernel(x, hyperedges, W1, W2, rv1, rv2, bn_gamma, bn_beta):
    heT = jnp.pad(hyperedges, ((0, EHP - EH), (0, 0))).T.reshape(-1)
    ones2d = jnp.ones((EPT // 128, 128), jnp.float32)
    zerosd = jnp.zeros((DHIST,), jnp.float32)
    g2d = (bn_gamma / jnp.sqrt(1.0 + 1e-5)).reshape(1, -1)
    b2d = bn_beta.reshape(1, -1)

    X1, proj1 = _matmul_proj(x, W1, rv1)
    u1, v1 = _uv_kernel(heT, proj1.reshape(-1))
    bd1, bs1, nb1 = _BIN1(u1, v1)
    cntr1 = _deg_kernel(u1.reshape(-1, 128), v1.reshape(-1, 128), ones2d, zerosd)
    Y1, dis1 = _SCALE128(X1, cntr1[:NP, None], cntr1[DHIST:DHIST + NP, None])
    zp1 = _ROW1(Y1, bd1, bs1, nb1)

    X2, proj2 = _bn_mm(zp1, dis1, g2d, b2d, W2, rv2)
    u2, v2 = _uv_kernel(heT, proj2.reshape(-1))
    bd2, bs2, nb2 = _BIN2(u2, v2)
    cntr2 = _deg_kernel(u2.reshape(-1, 128), v2.reshape(-1, 128), ones2d, zerosd)
    Y2, dis2 = _SCALE40(X2, cntr2[:NP, None], cntr2[DHIST:DHIST + NP, None])
    zp2 = _ROW2(Y2, bd2, bs2, nb2)

    return _lsm(zp2, dis2, W2.shape[1])


# speedup vs baseline: 1.0367x; 1.0001x over previous
import functools

import jax
import jax.numpy as jnp
from jax import lax
from jax.experimental import pallas as pl
from jax.experimental.pallas import tpu as pltpu
from jax.experimental.pallas import tpu_sc as plsc

N = 100000
NP = 102400
EH = 100000
EHP = 102400
EPT = EHP // 32
G = 96
GPAD = 128
CAPB = 6400
DUMP = NP
_SC_MESH = dict(core_axis_name="c", subcore_axis_name="s")


def _scalar_lane(vec, i):
    return jnp.sum(jnp.where(lax.iota(jnp.int32, 16) == i, vec, jnp.int32(0)))


def _uv_body(heT, proj_hbm, u_hbm, v_hbm, proj_v, e_vs, u_v, v_v):
    wid = lax.axis_index("c") * 16 + lax.axis_index("s")
    base = wid * EPT
    pltpu.sync_copy(proj_hbm.at[pl.ds(0, N)], proj_v)
    for k in range(4):
        pltpu.sync_copy(heT.at[pl.ds(k * EHP + base, EPT)], e_vs[k])

    def body(i, carry):
        off = i * 16
        e0 = e_vs[0][pl.ds(off, 16)]
        p0 = plsc.load_gather(proj_v, [e0])
        ubest, pmax = e0, p0
        vbest, pmin = e0, p0
        for k in range(1, 4):
            ek = e_vs[k][pl.ds(off, 16)]
            pk = plsc.load_gather(proj_v, [ek])
            mx = pk > pmax
            ubest = jnp.where(mx, ek, ubest)
            pmax = jnp.where(mx, pk, pmax)
            mn = pk < pmin
            vbest = jnp.where(mn, ek, vbest)
            pmin = jnp.where(mn, pk, pmin)
        valid = (base + off + lax.iota(jnp.int32, 16)) < EH
        dumpv = DUMP + lax.iota(jnp.int32, 16)
        u_v[pl.ds(off, 16)] = jnp.where(valid, ubest, dumpv)
        v_v[pl.ds(off, 16)] = jnp.where(valid, vbest, dumpv)
        return carry

    lax.fori_loop(0, EPT // 16, body, 0)
    pltpu.sync_copy(u_v, u_hbm.at[pl.ds(base, EPT)])
    pltpu.sync_copy(v_v, v_hbm.at[pl.ds(base, EPT)])


@functools.partial(
    pl.kernel,
    out_type=[
        jax.ShapeDtypeStruct((EHP,), jnp.int32),
        jax.ShapeDtypeStruct((EHP,), jnp.int32),
    ],
    mesh=plsc.VectorSubcoreMesh(**_SC_MESH),
    scratch_types=[
        pltpu.VMEM((N,), jnp.float32),
        [pltpu.VMEM((EPT,), jnp.int32)] * 4,
        pltpu.VMEM((EPT,), jnp.int32),
        pltpu.VMEM((EPT,), jnp.int32),
    ],
    compiler_params=pltpu.CompilerParams(needs_layout_passes=False, use_tc_tiling_on_sc=False),
)
def _uv_kernel(heT, proj, u_out, v_out, proj_v, e_vs, u_v, v_v):
    _uv_body(heT, proj, u_out, v_out, proj_v, e_vs, u_v, v_v)


def _make_bin_kernel(nchunk, chrows):
    def body(u_hbm, v_hbm, bd_hbm, bs_hbm, nb_hbm, u_v, v_v, bd_vs, bs_vs, nb_v):
        wid = lax.axis_index("c") * 16 + lax.axis_index("s")
        base = wid * EPT
        pltpu.sync_copy(u_hbm.at[pl.ds(base, EPT)], u_v)
        pltpu.sync_copy(v_hbm.at[pl.ds(base, EPT)], v_v)
        iot = lax.iota(jnp.int32, 16)

        def it(i, cnts):
            off = i * 16
            uu = u_v[pl.ds(off, 16)]
            vv = v_v[pl.ds(off, 16)]
            cnts = list(cnts)
            for dd, ss in ((vv, uu), (uu, vv)):
                for c in range(nchunk):
                    lo = c * chrows
                    m = (dd >= lo) & (dd < lo + chrows)
                    mi = m.astype(jnp.int32)
                    incl = plsc.cumsum(mi)
                    pos = cnts[c] + incl - mi
                    plsc.store_scatter(bd_vs[c], [pos], dd - lo, mask=m)
                    plsc.store_scatter(bs_vs[c], [pos], ss, mask=m)
                    cnts[c] = cnts[c] + jnp.max(incl)
            return tuple(cnts)

        cnts = lax.fori_loop(0, EPT // 16, it, (jnp.int32(0),) * nchunk)
        nbvec = jnp.zeros((16,), jnp.int32)
        for c in range(nchunk):
            for k in range(GPAD // 16):
                pos = cnts[c] + k * 16 + iot
                plsc.store_scatter(bd_vs[c], [pos], chrows + iot)
                plsc.store_scatter(bs_vs[c], [pos], wid * GPAD + k * 16 + iot)
            nbvec = jnp.where(iot == c, cnts[c], nbvec)
        nb_v[pl.ds(0, 16)] = nbvec
        pltpu.sync_copy(nb_v, nb_hbm.at[pl.ds(wid * 16, 16)])
        for c in range(nchunk):
            pltpu.sync_copy(bd_vs[c], bd_hbm.at[pl.ds((c * 32 + wid) * CAPB, CAPB)])
            pltpu.sync_copy(bs_vs[c], bs_hbm.at[pl.ds((c * 32 + wid) * CAPB, CAPB)])

    return pl.kernel(
        body,
        out_type=[
            jax.ShapeDtypeStruct((nchunk * 32 * CAPB,), jnp.int32),
            jax.ShapeDtypeStruct((nchunk * 32 * CAPB,), jnp.int32),
            jax.ShapeDtypeStruct((512,), jnp.int32),
        ],
        mesh=plsc.VectorSubcoreMesh(**_SC_MESH),
        scratch_types=[
            pltpu.VMEM((EPT,), jnp.int32),
            pltpu.VMEM((EPT,), jnp.int32),
            [pltpu.VMEM((CAPB,), jnp.int32)] * nchunk,
            [pltpu.VMEM((CAPB,), jnp.int32)] * nchunk,
            pltpu.VMEM((16,), jnp.int32),
        ],
        compiler_params=pltpu.CompilerParams(needs_layout_passes=False, use_tc_tiling_on_sc=False),
    )


DHIST = 102656
_DSTRIPE = DHIST // 16


def _deg_body(u2_hbm, v2_hbm, ones_hbm, zeros_hbm, cnt_hbm, hist_sh, ub2, vb2, ones_b, sem):
    cid = lax.axis_index("c")
    sid = lax.axis_index("s")
    wid = cid * 16 + sid
    rows = EPT // 128
    pltpu.sync_copy(zeros_hbm.at[pl.ds(sid * _DSTRIPE, _DSTRIPE)],
                    hist_sh.at[pl.ds(sid * _DSTRIPE, _DSTRIPE)])
    pltpu.sync_copy(ones_hbm, ones_b)
    pltpu.sync_copy(u2_hbm.at[pl.ds(wid * rows, rows)], ub2)
    pltpu.sync_copy(v2_hbm.at[pl.ds(wid * rows, rows)], vb2)
    plsc.subcore_barrier()
    hs = []
    for b in range(rows):
        hs.append(pltpu.async_copy(ones_b.at[b], hist_sh.at[ub2.at[b]], sem, add=True))
        hs.append(pltpu.async_copy(ones_b.at[b], hist_sh.at[vb2.at[b]], sem, add=True))
    for h in hs:
        h.wait()
    plsc.subcore_barrier()
    pltpu.sync_copy(hist_sh.at[pl.ds(sid * _DSTRIPE, _DSTRIPE)],
                    cnt_hbm.at[pl.ds(cid * DHIST + sid * _DSTRIPE, _DSTRIPE)])


_deg_kernel = pl.kernel(
    _deg_body,
    out_type=jax.ShapeDtypeStruct((2 * DHIST,), jnp.float32),
    mesh=plsc.VectorSubcoreMesh(**_SC_MESH),
    scratch_types=[
        pltpu.VMEM_SHARED((DHIST,), jnp.float32),
        pltpu.VMEM((EPT // 128, 128), jnp.int32),
        pltpu.VMEM((EPT // 128, 128), jnp.int32),
        pltpu.VMEM((EPT // 128, 128), jnp.float32),
        pltpu.SemaphoreType.DMA,
    ],
    compiler_params=pltpu.CompilerParams(needs_layout_passes=False, use_tc_tiling_on_sc=False),
)


def _make_row_kernel(nchunk, chrows, cpc, roww, tc_tiling=False, g=G):
    accr = chrows + 16
    sr = chrows // 16

    def body(y_hbm, bd_hbm, bs_hbm, nb_hbm, zp_hbm, acc_sh,
             db3, sb3, rows3, cbuf, sg):
        cid = lax.axis_index("c")
        sid = lax.axis_index("s")

        def seg_loop(c, w):
            pltpu.sync_copy(nb_hbm.at[pl.ds(w * 16, 16)], cbuf)
            cnt = _scalar_lane(cbuf[pl.ds(0, 16)], c)
            seg = (c * 32 + w) * CAPB

            def step(b, carry):
                par = lax.rem(b, 2)

                @pl.when(b * g < cnt)
                def _():
                    pltpu.sync_copy(bd_hbm.at[pl.ds(seg + b * g, g)], db3.at[par])
                    pltpu.sync_copy(bs_hbm.at[pl.ds(seg + b * g, g)], sb3.at[par])
                    pltpu.async_copy(y_hbm.at[sb3.at[par]], rows3.at[par], sg)

                @pl.when((b > 0) & ((b - 1) * g < cnt))
                def _():
                    q = 1 - par
                    pltpu.make_async_copy(y_hbm.at[sb3.at[q]], rows3.at[q], sg).wait()
                    pltpu.sync_copy(rows3.at[q], acc_sh.at[db3.at[q]], add=True)

                return carry

            lax.fori_loop(0, lax.shift_right_logical(cnt, 6) + 2, step, 0)

        for kk in range(cpc):
            c = cid * cpc + kk
            pltpu.sync_copy(y_hbm.at[pl.ds(c * chrows + sid * sr, sr)],
                            acc_sh.at[pl.ds(sid * sr, sr)])
            plsc.subcore_barrier()
            for j in range(2):
                seg_loop(c, 2 * sid + j)
            plsc.subcore_barrier()
            pltpu.sync_copy(acc_sh.at[pl.ds(sid * sr, sr)],
                            zp_hbm.at[pl.ds(c * chrows + sid * sr, sr)])

    return pl.kernel(
        body,
        out_type=jax.ShapeDtypeStruct((NP, roww), jnp.float32),
        mesh=plsc.VectorSubcoreMesh(**_SC_MESH),
        scratch_types=[
            pltpu.VMEM_SHARED((accr, roww), jnp.float32),
            pltpu.VMEM((2, g), jnp.int32),
            pltpu.VMEM((2, g), jnp.int32),
            pltpu.VMEM((2, g, roww), jnp.float32),
            pltpu.VMEM((16,), jnp.int32),
            pltpu.SemaphoreType.DMA,
        ],
        compiler_params=pltpu.CompilerParams(
            needs_layout_passes=False, use_tc_tiling_on_sc=tc_tiling),
    )


def _mm_proj_body(x_ref, W_ref, rv_ref, X_ref, p_ref):
    X = jnp.dot(x_ref[...], W_ref[...], preferred_element_type=jnp.float32)
    X_ref[...] = X
    p_ref[...] = jnp.dot(X, rv_ref[...], preferred_element_type=jnp.float32)


def _matmul_proj(x, W, rv, Bn=800):
    n, d = x.shape
    h = W.shape[1]
    return pl.pallas_call(
        _mm_proj_body,
        grid=(n // Bn,),
        in_specs=[
            pl.BlockSpec((Bn, d), lambda i: (i, 0)),
            pl.BlockSpec((d, h), lambda i: (0, 0)),
            pl.BlockSpec((h, 1), lambda i: (0, 0)),
        ],
        out_specs=[
            pl.BlockSpec((Bn, h), lambda i: (i, 0)),
            pl.BlockSpec((Bn, 1), lambda i: (i, 0)),
        ],
        out_shape=[
            jax.ShapeDtypeStruct((NP, h), jnp.float32),
            jax.ShapeDtypeStruct((NP, 1), jnp.float32),
        ],
    )(x, W, rv)


def _scale_body(x_ref, ca_ref, cb_ref, y_ref, dis_ref):
    dis = lax.rsqrt(jnp.maximum(ca_ref[...] + cb_ref[...] + 1.0, 1e-12))
    dis_ref[...] = dis
    y_ref[...] = x_ref[...] * dis


def _make_scale_kernel(roww, Bn=2048):
    return pl.pallas_call(
        _scale_body,
        grid=(NP // Bn,),
        in_specs=[
            pl.BlockSpec((Bn, roww), lambda i: (i, 0)),
            pl.BlockSpec((Bn, 1), lambda i: (i, 0)),
            pl.BlockSpec((Bn, 1), lambda i: (i, 0)),
        ],
        out_specs=[
            pl.BlockSpec((Bn, roww), lambda i: (i, 0)),
            pl.BlockSpec((Bn, 1), lambda i: (i, 0)),
        ],
        out_shape=[
            jax.ShapeDtypeStruct((NP, roww), jnp.float32),
            jax.ShapeDtypeStruct((NP, 1), jnp.float32),
        ],
    )


def _bn_mm_body(zp_ref, dis_ref, g_ref, b_ref, W_ref, rv_ref, x2_ref, p2_ref):
    t = jnp.maximum(zp_ref[...] * dis_ref[...], 0.0)
    t = t * g_ref[...] + b_ref[...]
    X2 = jnp.dot(t, W_ref[...], preferred_element_type=jnp.float32)
    x2_ref[...] = X2
    p2_ref[...] = jnp.dot(X2, rv_ref[...], preferred_element_type=jnp.float32)


def _bn_mm(zp, dis, g2d, b2d, W2, rv2, Bn=2048):
    h, c = W2.shape
    return pl.pallas_call(
        _bn_mm_body,
        grid=(NP // Bn,),
        in_specs=[
            pl.BlockSpec((Bn, h), lambda i: (i, 0)),
            pl.BlockSpec((Bn, 1), lambda i: (i, 0)),
            pl.BlockSpec((1, h), lambda i: (0, 0)),
            pl.BlockSpec((1, h), lambda i: (0, 0)),
            pl.BlockSpec((h, c), lambda i: (0, 0)),
            pl.BlockSpec((c, 1), lambda i: (0, 0)),
        ],
        out_specs=[
            pl.BlockSpec((Bn, c), lambda i: (i, 0)),
            pl.BlockSpec((Bn, 1), lambda i: (i, 0)),
        ],
        out_shape=[
            jax.ShapeDtypeStruct((NP, c), jnp.float32),
            jax.ShapeDtypeStruct((NP, 1), jnp.float32),
        ],
    )(zp, dis, g2d, b2d, W2, rv2)


def _lsm_body(zp_ref, dis_ref, out_ref):
    L = zp_ref[...] * dis_ref[...]
    m = jnp.max(L, axis=-1, keepdims=True)
    s = jnp.log(jnp.sum(jnp.exp(L - m), axis=-1, keepdims=True))
    out_ref[...] = L - m - s


def _lsm(zp, dis, c, Bn=800):
    return pl.pallas_call(
        _lsm_body,
        grid=(N // Bn,),
        in_specs=[
            pl.BlockSpec((Bn, c), lambda i: (i, 0)),
            pl.BlockSpec((Bn, 1), lambda i: (i, 0)),
        ],
        out_specs=pl.BlockSpec((Bn, c), lambda i: (i, 0)),
        out_shape=jax.ShapeDtypeStruct((N, c), jnp.float32),
    )(zp, dis)


_L1 = dict(nchunk=8, chrows=12704, cpc=4)
_L2 = dict(nchunk=4, chrows=25600, cpc=2)
_BIN1 = _make_bin_kernel(_L1["nchunk"], _L1["chrows"])
_BIN2 = _make_bin_kernel(_L2["nchunk"], _L2["chrows"])
_ROW1 = _make_row_kernel(roww=128, g=112, **_L1)
_ROW2 = _make_row_kernel(roww=40, g=128, **_L2)
_SCALE128 = _make_scale_kernel(128)
_SCALE40 = _make_scale_kernel(40)


def kernel(x, hyperedges, W1, W2, rv1, rv2, bn_gamma, bn_beta):
    heT = jnp.pad(hyperedges, ((0, EHP - EH), (0, 0))).T.reshape(-1)
    ones2d = jnp.ones((EPT // 128, 128), jnp.float32)
    zerosd = jnp.zeros((DHIST,), jnp.float32)
    g2d = (bn_gamma / jnp.sqrt(1.0 + 1e-5)).reshape(1, -1)
    b2d = bn_beta.reshape(1, -1)

    X1, proj1 = _matmul_proj(x, W1, rv1)
    u1, v1 = _uv_kernel(heT, proj1.reshape(-1))
    bd1, bs1, nb1 = _BIN1(u1, v1)
    cntr1 = _deg_kernel(u1.reshape(-1, 128), v1.reshape(-1, 128), ones2d, zerosd)
    Y1, dis1 = _SCALE128(X1, cntr1[:NP, None], cntr1[DHIST:DHIST + NP, None])
    zp1 = _ROW1(Y1, bd1, bs1, nb1)

    X2, proj2 = _bn_mm(zp1, dis1, g2d, b2d, W2, rv2)
    u2, v2 = _uv_kernel(heT, proj2.reshape(-1))
    bd2, bs2, nb2 = _BIN2(u2, v2)
    cntr2 = _deg_kernel(u2.reshape(-1, 128), v2.reshape(-1, 128), ones2d, zerosd)
    Y2, dis2 = _SCALE40(X2, cntr2[:NP, None], cntr2[DHIST:DHIST + NP, None])
    zp2 = _ROW2(Y2, bd2, bs2, nb2)

    return _lsm(zp2, dis2, W2.shape[1])
